# baseline (speedup 1.0000x reference)
import functools

import jax
import jax.numpy as jnp
from jax import lax
from jax.experimental import pallas as pl
from jax.experimental.pallas import tpu as pltpu
from jax.experimental.pallas import tpu_sc as plsc

D_MODEL = 64
SCALE = 8.0
LANES = 16
NW = 32
Q = 128


def _emb_kernel(xT_hbm, lut_hbm, out_hbm, idx_all, rot_tbl,
                g0, g1, m0, m1, t0, t1, gsem0, gsem1, ssem0, ssem1, *, nj):
    w = lax.axis_index("s") * 2 + lax.axis_index("c")
    i0 = w * Q
    pairs = nj // 2

    bufs = ((g0, m0, t0, gsem0, ssem0), (g1, m1, t1, gsem1, ssem1))

    pltpu.sync_copy(xT_hbm.at[:, pl.ds(i0, Q)], idx_all)

    def fire_gather(j, buf):
        gb, mr, _, gsem, _ = buf
        for k in range(Q // LANES):
            sl = pl.ds(LANES * k, LANES)
            mr[sl] = lax.shift_right_logical(idx_all[j, sl], 1)
        pltpu.async_copy(lut_hbm.at[mr], gb, gsem)

    def drain_gather(buf):
        gb, _, _, gsem, _ = buf
        pltpu.make_async_copy(lut_hbm.at[pl.ds(0, Q)], gb, gsem).wait()

    def fire_store(j, buf):
        _, _, tb, _, ssem = buf
        pltpu.async_copy(tb, out_hbm.at[j, :, pl.ds(i0, Q)], ssem)

    def drain_store(buf):
        _, _, tb, _, ssem = buf
        pltpu.make_async_copy(tb, out_hbm.at[0, :, pl.ds(i0, Q)], ssem).wait()

    iota = jax.lax.iota(jnp.int32, LANES)
    ridx = [iota + LANES * k for k in range(Q // LANES)]
    ones = jnp.full((LANES,), 1, jnp.int32)
    fifteen = jnp.full((LANES,), LANES - 1, jnp.int32)

    for t in range(LANES):
        rot_tbl[t, :] = (iota + t) & fifteen

    def transpose_scale(j, buf):
        gb, _, tb, _, _ = buf
        for k in range(Q // LANES):
            sl = pl.ds(LANES * k, LANES)
            h = lax.shift_left(idx_all[j, sl] & ones, 6)
            rk = ridx[k]

            @plsc.parallel_loop(0, D_MODEL, step=1, unroll=4)
            def _(i):
                t = i & (LANES - 1)
                db16 = i - t
                rot = rot_tbl[t, :]
                v = plsc.load_gather(gb, [rk, (h + db16) + rot])
                plsc.store_scatter(tb, [rot + db16, rk], v * SCALE)

    for b in (0, 1):
        fire_gather(b, bufs[b])

    def body(p, _):
        for b in (0, 1):
            j = 2 * p + b
            buf = bufs[b]
            drain_gather(buf)

            @pl.when(p > 0)
            def _():
                drain_store(buf)

            transpose_scale(j, buf)
            fire_store(j, buf)

            @pl.when(j + 2 < nj)
            def _():
                fire_gather(j + 2, buf)

        return 0

    lax.fori_loop(0, pairs, body, 0)

    for b in (0, 1):
        drain_store(bufs[b])


def kernel(x, lut):
    s0, s1 = x.shape
    xT = x.T.astype(jnp.int32)
    lut_m = lut.reshape(lut.shape[0] // 2, 2 * D_MODEL)

    mesh = plsc.VectorSubcoreMesh(core_axis_name="c", subcore_axis_name="s")
    k = pl.kernel(
        functools.partial(_emb_kernel, nj=s1),
        mesh=mesh,
        out_type=jax.ShapeDtypeStruct((s1, D_MODEL, s0), jnp.float32),
        scratch_types=[
            pltpu.VMEM((s1, Q), jnp.int32),
            pltpu.VMEM((LANES, LANES), jnp.int32),
            pltpu.VMEM((Q, Q), jnp.float32),
            pltpu.VMEM((Q, Q), jnp.float32),
            pltpu.VMEM((Q,), jnp.int32),
            pltpu.VMEM((Q,), jnp.int32),
            pltpu.VMEM((D_MODEL, Q), jnp.float32),
            pltpu.VMEM((D_MODEL, Q), jnp.float32),
            pltpu.SemaphoreType.DMA,
            pltpu.SemaphoreType.DMA,
            pltpu.SemaphoreType.DMA,
            pltpu.SemaphoreType.DMA,
        ],
        compiler_params=pltpu.CompilerParams(
            use_tc_tiling_on_sc=True, needs_layout_passes=False
        ),
    )
    out = k(xT, lut_m)
    return jnp.transpose(out, (2, 0, 1))


# speedup vs baseline: 1.0451x; 1.0451x over previous
import functools

import jax
import jax.numpy as jnp
from jax import lax
from jax.experimental import pallas as pl
from jax.experimental.pallas import tpu as pltpu
from jax.experimental.pallas import tpu_sc as plsc

D_MODEL = 64
SCALE = 8.0
LANES = 16
NW = 32
Q = 128
NG = 4
NS = 2


def _emb_kernel(xT_hbm, lut_hbm, out_hbm, idx_all, rot_tbl,
                gbs, mrs, tbs, gsems, ssems, *, nj):
    w = lax.axis_index("s") * 2 + lax.axis_index("c")
    i0 = w * Q

    pltpu.sync_copy(xT_hbm.at[:, pl.ds(i0, Q)], idx_all)

    iota = jax.lax.iota(jnp.int32, LANES)
    ridx = [iota + LANES * k for k in range(Q // LANES)]
    ones = jnp.full((LANES,), 1, jnp.int32)
    fifteen = jnp.full((LANES,), LANES - 1, jnp.int32)

    for t in range(LANES):
        rot_tbl[t, :] = (iota + t) & fifteen

    def fire_gather(j, b):
        mr = mrs[b]
        for k in range(Q // LANES):
            sl = pl.ds(LANES * k, LANES)
            mr[sl] = lax.shift_right_logical(idx_all[j, sl], 1)
        pltpu.async_copy(lut_hbm.at[mr], gbs[b], gsems[b])

    def drain_gather(b):
        pltpu.make_async_copy(lut_hbm.at[pl.ds(0, Q)], gbs[b], gsems[b]).wait()

    def fire_store(j, s):
        pltpu.async_copy(tbs[s], out_hbm.at[j, :, pl.ds(i0, Q)], ssems[s])

    def drain_store(s):
        pltpu.make_async_copy(
            tbs[s], out_hbm.at[0, :, pl.ds(i0, Q)], ssems[s]
        ).wait()

    def transpose_scale(j, b, s):
        gb, tb = gbs[b], tbs[s]
        for k in range(Q // LANES):
            sl = pl.ds(LANES * k, LANES)
            h = lax.shift_left(idx_all[j, sl] & ones, 6)
            rk = ridx[k]

            @plsc.parallel_loop(0, D_MODEL, step=1, unroll=4)
            def _(i):
                t = i & (LANES - 1)
                db16 = i - t
                rot = rot_tbl[t, :] + db16
                v = plsc.load_gather(gb, [rk, h + rot])
                plsc.store_scatter(tb, [rot, rk], v * SCALE)

    for b in range(NG):
        fire_gather(b, b)

    def body(p, _):
        for b in range(NG):
            j = NG * p + b
            s = b % NS
            drain_gather(b)

            @pl.when(j >= NS)
            def _():
                drain_store(s)

            transpose_scale(j, b, s)
            fire_store(j, s)

            @pl.when(j + NG < nj)
            def _():
                fire_gather(j + NG, b)

        return 0

    lax.fori_loop(0, nj // NG, body, 0)

    for s in range(NS):
        drain_store(s)


def kernel(x, lut):
    s0, s1 = x.shape
    xT = x.T.astype(jnp.int32)
    lut_m = lut.reshape(lut.shape[0] // 2, 2 * D_MODEL)

    mesh = plsc.VectorSubcoreMesh(core_axis_name="c", subcore_axis_name="s")
    k = pl.kernel(
        functools.partial(_emb_kernel, nj=s1),
        mesh=mesh,
        out_type=jax.ShapeDtypeStruct((s1, D_MODEL, s0), jnp.float32),
        scratch_types=[
            pltpu.VMEM((s1, Q), jnp.int32),
            pltpu.VMEM((LANES, LANES), jnp.int32),
            [pltpu.VMEM((Q, Q), jnp.float32) for _ in range(NG)],
            [pltpu.VMEM((Q,), jnp.int32) for _ in range(NG)],
            [pltpu.VMEM((D_MODEL, Q), jnp.float32) for _ in range(NS)],
            [pltpu.SemaphoreType.DMA for _ in range(NG)],
            [pltpu.SemaphoreType.DMA for _ in range(NS)],
        ],
        compiler_params=pltpu.CompilerParams(
            use_tc_tiling_on_sc=True, needs_layout_passes=False
        ),
    )
    out = k(xT, lut_m)
    return jnp.transpose(out, (2, 0, 1))


# speedup vs baseline: 2.1193x; 2.0280x over previous
import functools

import jax
import jax.numpy as jnp
from jax import lax
from jax.experimental import pallas as pl
from jax.experimental.pallas import tpu as pltpu
from jax.experimental.pallas import tpu_sc as plsc

D_MODEL = 64
SCALE = 8.0
LANES = 16
NW = 32
Q = 128
NG = 4
NS = 2


def _emb_kernel(xT_hbm, lut_hbm, out_hbm, idx_all, rot_tbl,
                gbs, mrs, tbs, gsems, ssems, *, nj):
    w = lax.axis_index("s") * 2 + lax.axis_index("c")
    i0 = w * Q

    pltpu.sync_copy(xT_hbm.at[:, pl.ds(i0, Q)], idx_all)

    iota = jax.lax.iota(jnp.int32, LANES)
    ridx = [iota + LANES * k for k in range(Q // LANES)]
    ones = jnp.full((LANES,), 1, jnp.int32)
    fifteen = jnp.full((LANES,), LANES - 1, jnp.int32)

    for t in range(LANES):
        rot_tbl[t, :] = (iota + t) & fifteen

    def fire_gather(j, b):
        mr = mrs[b]
        for k in range(Q // LANES):
            sl = pl.ds(LANES * k, LANES)
            mr[sl] = lax.shift_right_logical(idx_all[j, sl], 1)
        pltpu.async_copy(lut_hbm.at[mr], gbs[b], gsems[b])

    def drain_gather(b):
        pltpu.make_async_copy(lut_hbm.at[pl.ds(0, Q)], gbs[b], gsems[b]).wait()

    def fire_store(j, s):
        pltpu.async_copy(tbs[s], out_hbm.at[j, :, pl.ds(i0, Q)], ssems[s])

    def drain_store(s):
        pltpu.make_async_copy(
            tbs[s], out_hbm.at[0, :, pl.ds(i0, Q)], ssems[s]
        ).wait()

    def transpose_scale(j, b, s):
        gb, tb = gbs[b], tbs[s]
        for k in range(Q // LANES):
            sl = pl.ds(LANES * k, LANES)
            h = lax.shift_left(idx_all[j, sl] & ones, 6)
            rk = ridx[k]

            @plsc.parallel_loop(0, D_MODEL, step=1, unroll=4)
            def _(i):
                t = i & (LANES - 1)
                db16 = i - t
                rot = rot_tbl[t, :] + db16
                v = plsc.load_gather(gb, [rk, h + rot])
                plsc.store_scatter(tb, [rot, rk], v * SCALE)

    for b in range(NG):
        fire_gather(b, b)

    def body(p, _):
        for b in range(NG):
            j = NG * p + b
            s = b % NS
            drain_gather(b)

            @pl.when(j >= NS)
            def _():
                drain_store(s)

            transpose_scale(j, b, s)
            fire_store(j, s)

            @pl.when(j + NG < nj)
            def _():
                fire_gather(j + NG, b)

        return 0

    lax.fori_loop(0, nj // NG, body, 0)

    for s in range(NS):
        drain_store(s)


def _merge_kernel(lutT_hbm, out_hbm, rot_tbl, vbs, obs, tvb, rsems, wsems, *, v):
    w = lax.axis_index("s") * 2 + lax.axis_index("c")
    n_full = (v // 2) // D_MODEL
    iters = n_full // NW + 1

    iota = jax.lax.iota(jnp.int32, LANES)
    fifteen = jnp.full((LANES,), LANES - 1, jnp.int32)
    for t in range(LANES):
        rot_tbl[t, :] = (iota + t) & fifteen
    iota2 = iota * 2

    def transpose_block(vb, ob, n_a):
        @plsc.parallel_loop(0, n_a * 128, step=1, unroll=4)
        def _(i):
            t = i & (LANES - 1)
            b = (i >> 4) & 7
            a = i >> 7
            rot = rot_tbl[t, :]
            rows = rot + (b & 3) * LANES
            cols = iota2 + (a * 32 + (b >> 2))
            val = plsc.load_gather(vb, [rows, cols])
            plsc.store_scatter(ob, [iota + a * LANES, rot + b * LANES], val)

    def fire_read(c, r):
        pltpu.async_copy(lutT_hbm.at[:, pl.ds(c * Q, Q)], vbs[r], rsems[r])

    def drain_read(r):
        pltpu.make_async_copy(
            lutT_hbm.at[:, pl.ds(0, Q)], vbs[r], rsems[r]
        ).wait()

    def fire_write(c, r):
        pltpu.async_copy(obs[r], out_hbm.at[pl.ds(c * D_MODEL, D_MODEL)], wsems[r])

    def drain_write(r):
        pltpu.make_async_copy(
            obs[r], out_hbm.at[pl.ds(0, D_MODEL)], wsems[r]
        ).wait()

    c0 = w

    @pl.when(c0 < n_full)
    def _():
        fire_read(c0, 0)

    @pl.when(c0 + NW < n_full)
    def _():
        fire_read(c0 + NW, 1)

    def body(i, _):
        for r in (0, 1):
            c = c0 + NW * (2 * i + r)

            @pl.when(c < n_full)
            def _():
                drain_read(r)

                @pl.when(2 * i + r >= 2)
                def _():
                    drain_write(r)

                transpose_block(vbs[r], obs[r], 4)
                fire_write(c, r)

                @pl.when(c + 2 * NW < n_full)
                def _():
                    fire_read(c + 2 * NW, r)

        return 0

    lax.fori_loop(0, (iters + 1) // 2, body, 0)

    for r in (0, 1):
        @pl.when(c0 + NW * r < n_full)
        def _():
            drain_write(r)

    @pl.when(w == NW - 1)
    def _():
        pltpu.sync_copy(lutT_hbm.at[:, pl.ds(n_full * Q, D_MODEL)], tvb)
        transpose_block(tvb, obs[0], 2)
        pltpu.sync_copy(
            obs[0].at[pl.ds(0, 32)], out_hbm.at[pl.ds(n_full * D_MODEL, 32)]
        )


def _merge_lut(lut):
    v = lut.shape[0]
    mesh = plsc.VectorSubcoreMesh(core_axis_name="c", subcore_axis_name="s")
    k = pl.kernel(
        functools.partial(_merge_kernel, v=v),
        mesh=mesh,
        out_type=jax.ShapeDtypeStruct((v // 2, 2 * D_MODEL), jnp.float32),
        scratch_types=[
            pltpu.VMEM((LANES, LANES), jnp.int32),
            [pltpu.VMEM((D_MODEL, Q), jnp.float32) for _ in range(2)],
            [pltpu.VMEM((D_MODEL, Q), jnp.float32) for _ in range(2)],
            pltpu.VMEM((D_MODEL, D_MODEL), jnp.float32),
            [pltpu.SemaphoreType.DMA for _ in range(2)],
            [pltpu.SemaphoreType.DMA for _ in range(2)],
        ],
        compiler_params=pltpu.CompilerParams(
            use_tc_tiling_on_sc=True, needs_layout_passes=False
        ),
    )
    return k(lut.T)


def kernel(x, lut):
    s0, s1 = x.shape
    xT = x.T.astype(jnp.int32)
    lut_m = _merge_lut(lut)

    mesh = plsc.VectorSubcoreMesh(core_axis_name="c", subcore_axis_name="s")
    k = pl.kernel(
        functools.partial(_emb_kernel, nj=s1),
        mesh=mesh,
        out_type=jax.ShapeDtypeStruct((s1, D_MODEL, s0), jnp.float32),
        scratch_types=[
            pltpu.VMEM((s1, Q), jnp.int32),
            pltpu.VMEM((LANES, LANES), jnp.int32),
            [pltpu.VMEM((Q, Q), jnp.float32) for _ in range(NG)],
            [pltpu.VMEM((Q,), jnp.int32) for _ in range(NG)],
            [pltpu.VMEM((D_MODEL, Q), jnp.float32) for _ in range(NS)],
            [pltpu.SemaphoreType.DMA for _ in range(NG)],
            [pltpu.SemaphoreType.DMA for _ in range(NS)],
        ],
        compiler_params=pltpu.CompilerParams(
            use_tc_tiling_on_sc=True, needs_layout_passes=False
        ),
    )
    out = k(xT, lut_m)
    return jnp.transpose(out, (2, 0, 1))
